# baseline (speedup 1.0000x reference)
import jax
import jax.numpy as jnp
from jax.experimental import pallas as pl
from jax.experimental.pallas import tpu as pltpu

HIDDEN_DIM = 2048
NUM_EXPERTS = 8
EXPERT_DIM = 4096

M_BLK = 1024
E_BLK = 512
LANE = 128


def _moe_body(x_ref, wr_ref, wg_ref, wu_ref, wd_ref, out_ref, s_ref):
    e = pl.program_id(1)
    n_e = pl.num_programs(1)
    xb = x_ref[...]

    @pl.when(e == 0)
    def _router():
        logits = jax.lax.dot_general(
            xb, wr_ref[...], (((1,), (0,)), ((), ())),
            preferred_element_type=jnp.float32)
        lane = jax.lax.broadcasted_iota(jnp.int32, logits.shape, 1)
        valid = lane < NUM_EXPERTS
        neg_inf = jnp.float32(-jnp.inf)
        logits = jnp.where(valid, logits, neg_inf)
        m1 = jnp.max(logits, axis=1, keepdims=True)
        eq = jnp.logical_and(logits == m1, valid)
        cnt = jnp.sum(eq.astype(jnp.float32), axis=1, keepdims=True)
        masked = jnp.where(eq, neg_inf, logits)
        m2 = jnp.max(masked, axis=1, keepdims=True)
        l2 = jnp.where(cnt >= 2.0, m1, m2)
        z = jnp.sum(jnp.where(valid, jnp.exp(logits - m1), 0.0), axis=1, keepdims=True)
        s_ref[...] = (1.0 + jnp.exp(l2 - m1)) / z

    gate = jax.lax.dot_general(
        xb, wg_ref[...], (((1,), (0,)), ((), ())),
        preferred_element_type=jnp.float32)
    up = jax.lax.dot_general(
        xb, wu_ref[...], (((1,), (0,)), ((), ())),
        preferred_element_type=jnp.float32)
    act = (gate * jax.nn.sigmoid(gate) * up).astype(jnp.bfloat16)
    part = jax.lax.dot_general(
        act, wd_ref[...], (((1,), (0,)), ((), ())),
        preferred_element_type=jnp.float32)

    @pl.when(e == 0)
    def _init():
        out_ref[...] = part

    @pl.when(e > 0)
    def _acc():
        out_ref[...] += part

    @pl.when(e == n_e - 1)
    def _scale():
        out_ref[...] *= s_ref[...]


def kernel(x, W_router, W_gate, W_up, W_down):
    orig_shape = x.shape
    tokens = orig_shape[0] * orig_shape[1]
    xb = x.reshape(tokens, HIDDEN_DIM).astype(jnp.bfloat16)
    wr = jnp.pad(W_router.T, ((0, 0), (0, LANE - NUM_EXPERTS))).astype(jnp.bfloat16)
    wg = W_gate.T.astype(jnp.bfloat16)
    wu = W_up.T.astype(jnp.bfloat16)
    wd = W_down.T.astype(jnp.bfloat16)

    n_m = tokens // M_BLK
    n_e = EXPERT_DIM // E_BLK

    out = pl.pallas_call(
        _moe_body,
        grid=(n_m, n_e),
        in_specs=[
            pl.BlockSpec((M_BLK, HIDDEN_DIM), lambda m, e: (m, 0)),
            pl.BlockSpec((HIDDEN_DIM, LANE), lambda m, e: (0, 0)),
            pl.BlockSpec((HIDDEN_DIM, E_BLK), lambda m, e: (0, e)),
            pl.BlockSpec((HIDDEN_DIM, E_BLK), lambda m, e: (0, e)),
            pl.BlockSpec((E_BLK, HIDDEN_DIM), lambda m, e: (e, 0)),
        ],
        out_specs=pl.BlockSpec((M_BLK, HIDDEN_DIM), lambda m, e: (m, 0)),
        out_shape=jax.ShapeDtypeStruct((tokens, HIDDEN_DIM), jnp.float32),
        scratch_shapes=[pltpu.VMEM((M_BLK, 1), jnp.float32)],
    )(xb, wr, wg, wu, wd)
    return out.reshape(orig_shape)


# speedup vs baseline: 1.0256x; 1.0256x over previous
import jax
import jax.numpy as jnp
from jax.experimental import pallas as pl
from jax.experimental.pallas import tpu as pltpu

HIDDEN_DIM = 2048
NUM_EXPERTS = 8
EXPERT_DIM = 4096

M_BLK = 1024
E_BLK = 256

_DN_T = (((1,), (1,)), ((), ()))


def _moe_body(x_ref, wr_ref, wg_ref, wu_ref, wd_ref, out_ref, s_ref):
    e = pl.program_id(1)
    n_e = pl.num_programs(1)
    xb = x_ref[...].astype(jnp.bfloat16)

    @pl.when(e == 0)
    def _router():
        logits = jax.lax.dot_general(
            xb, wr_ref[...].astype(jnp.bfloat16), _DN_T,
            preferred_element_type=jnp.float32)
        neg_inf = jnp.float32(-jnp.inf)
        m1 = jnp.max(logits, axis=1, keepdims=True)
        eq = logits == m1
        cnt = jnp.sum(eq.astype(jnp.float32), axis=1, keepdims=True)
        m2 = jnp.max(jnp.where(eq, neg_inf, logits), axis=1, keepdims=True)
        l2 = jnp.where(cnt >= 2.0, m1, m2)
        z = jnp.sum(jnp.exp(logits - m1), axis=1, keepdims=True)
        s_ref[...] = (1.0 + jnp.exp(l2 - m1)) / z

    gate = jax.lax.dot_general(
        xb, wg_ref[...].astype(jnp.bfloat16), _DN_T,
        preferred_element_type=jnp.float32)
    up = jax.lax.dot_general(
        xb, wu_ref[...].astype(jnp.bfloat16), _DN_T,
        preferred_element_type=jnp.float32)
    act = (gate * jax.nn.sigmoid(gate) * up).astype(jnp.bfloat16)
    part = jax.lax.dot_general(
        act, wd_ref[...].astype(jnp.bfloat16), _DN_T,
        preferred_element_type=jnp.float32)

    @pl.when(e == 0)
    def _init():
        out_ref[...] = part

    @pl.when(e > 0)
    def _acc():
        out_ref[...] += part

    @pl.when(e == n_e - 1)
    def _scale():
        out_ref[...] *= s_ref[...]


def kernel(x, W_router, W_gate, W_up, W_down):
    orig_shape = x.shape
    tokens = orig_shape[0] * orig_shape[1]
    xf = x.reshape(tokens, HIDDEN_DIM)

    n_m = tokens // M_BLK
    n_e = EXPERT_DIM // E_BLK

    out = pl.pallas_call(
        _moe_body,
        grid=(n_m, n_e),
        in_specs=[
            pl.BlockSpec((M_BLK, HIDDEN_DIM), lambda m, e: (m, 0)),
            pl.BlockSpec((NUM_EXPERTS, HIDDEN_DIM), lambda m, e: (0, 0)),
            pl.BlockSpec((E_BLK, HIDDEN_DIM), lambda m, e: (e, 0)),
            pl.BlockSpec((E_BLK, HIDDEN_DIM), lambda m, e: (e, 0)),
            pl.BlockSpec((HIDDEN_DIM, E_BLK), lambda m, e: (0, e)),
        ],
        out_specs=pl.BlockSpec((M_BLK, HIDDEN_DIM), lambda m, e: (m, 0)),
        out_shape=jax.ShapeDtypeStruct((tokens, HIDDEN_DIM), jnp.float32),
        scratch_shapes=[pltpu.VMEM((M_BLK, 1), jnp.float32)],
    )(xf, W_router, W_gate, W_up, W_down)
    return out.reshape(orig_shape)
